# baseline (speedup 1.0000x reference)
import jax
import jax.numpy as jnp
from jax import lax
from jax.experimental import pallas as pl
from jax.experimental.pallas import tpu as pltpu
from jax.experimental.pallas import tpu_sc as plsc

N = 10000
NP = 10240
E = 640000
NW = 32
C = 256
K = 80
EPW = K * C
EPAD = NW * EPW
RPT = NP // 16
BR = 1280


def _mesh():
    return plsc.VectorSubcoreMesh(core_axis_name="c", subcore_axis_name="s")


def _deg_call(dsts, iden):
    def body(dsts_hbm, iden_hbm, part_hbm, dst_v, deg1_v, deg2_v, iden_v, shared):
        cid = lax.axis_index("c")
        sid = lax.axis_index("s")
        wid = sid * 2 + cid
        pltpu.sync_copy(dsts_hbm.at[wid], dst_v)
        pltpu.sync_copy(iden_hbm, iden_v)

        zero = jnp.zeros((16,), jnp.float32)

        def zbody(i, carry):
            deg1_v[pl.ds(i * 16, 16)] = zero
            return carry

        lax.fori_loop(0, NP // 16, zbody, 0)

        def z2body(t, carry):
            for s in range(8):
                deg2_v[t, pl.ds(s * 16, 16)] = zero
            return carry

        lax.fori_loop(0, 80, z2body, 0)

        @pl.when(sid == 0)
        def _():
            pltpu.sync_copy(deg2_v, shared)

        ones = jnp.ones((16,), jnp.float32)

        def ebody(j, carry):
            for s in range(C // 16):
                idx = dst_v[j, pl.ds(s * 16, 16)]
                plsc.addupdate_scatter(deg1_v, [idx], ones)
            return carry

        lax.fori_loop(0, K, ebody, 0)

        def pbody(t, carry):
            for s in range(8):
                deg2_v[t, pl.ds(s * 16, 16)] = deg1_v[pl.ds(t * 128 + s * 16, 16)]
            return carry

        lax.fori_loop(0, 80, pbody, 0)
        plsc.subcore_barrier()
        pltpu.sync_copy(deg2_v, shared.at[iden_v], add=True)
        plsc.subcore_barrier()

        @pl.when(sid < 10)
        def _():
            sl = pl.ds(sid * 8, 8)
            pltpu.sync_copy(shared.at[sl], part_hbm.at[cid, sl])

    return pl.kernel(
        body,
        out_type=jax.ShapeDtypeStruct((2, 80, 128), jnp.float32),
        mesh=_mesh(),
        compiler_params=pltpu.CompilerParams(
            needs_layout_passes=False, use_tc_tiling_on_sc=False),
        scratch_types=[
            pltpu.VMEM((K, C), jnp.int32),
            pltpu.VMEM((NP,), jnp.float32),
            pltpu.VMEM((80, 128), jnp.float32),
            pltpu.VMEM((80,), jnp.int32),
            pltpu.VMEM_SHARED((80, 128), jnp.float32),
        ],
    )(dsts, iden)


def _layer_call(F, NB, h, zeros, srcs, dsts):
    KB = K // NB

    def body(*refs):
        h_hbm, z_hbm, srcs_hbm, dsts_hbm, part_hbm, src_v, dst_v = refs[:7]
        bufs = refs[7:7 + 2 * NB]
        gsem = refs[7 + 2 * NB:9 + 2 * NB]
        ssem = refs[9 + 2 * NB:11 + 2 * NB]
        shared = refs[11 + 2 * NB]
        rows = (bufs[:NB], bufs[NB:])
        cid = lax.axis_index("c")
        sid = lax.axis_index("s")
        wid = sid * 2 + cid
        pltpu.sync_copy(srcs_hbm.at[wid], src_v)
        pltpu.sync_copy(dsts_hbm.at[wid], dst_v)
        sl = pl.ds(sid * RPT, RPT)

        @pl.when(cid == 0)
        def _():
            pltpu.sync_copy(h_hbm.at[sl], shared.at[sl])

        @pl.when(cid != 0)
        def _():
            pltpu.sync_copy(z_hbm.at[sl], shared.at[sl])

        plsc.subcore_barrier()

        dummy = h_hbm.at[pl.ds(0, C)]

        def start_gathers(t, p):
            for b in range(NB):
                pltpu.async_copy(h_hbm.at[src_v.at[t * NB + b]], rows[p][b], gsem[p])

        def fire_scatters(t, p):
            for b in range(NB):
                pltpu.async_copy(rows[p][b], shared.at[dst_v.at[t * NB + b]],
                                 ssem[p], add=True)

        def drain(sem, p):
            for b in range(NB):
                pltpu.make_async_copy(dummy, rows[p][b], sem).wait()

        start_gathers(0, 0)

        def phase(t, p):
            @pl.when(t > 0)
            def _():
                drain(ssem[1 - p], 1 - p)

            @pl.when(t + 1 < KB)
            def _():
                start_gathers(t + 1, 1 - p)

            drain(gsem[p], p)
            fire_scatters(t, p)

        def outer(i, carry):
            phase(2 * i, 0)
            phase(2 * i + 1, 1)
            return carry

        lax.fori_loop(0, KB // 2, outer, 0)
        drain(ssem[(KB - 1) % 2], (KB - 1) % 2)
        plsc.subcore_barrier()
        pltpu.sync_copy(shared.at[sl], part_hbm.at[cid, sl])

    return pl.kernel(
        body,
        out_type=jax.ShapeDtypeStruct((2, NP, F), jnp.float32),
        mesh=_mesh(),
        compiler_params=pltpu.CompilerParams(use_tc_tiling_on_sc=False),
        scratch_types=(
            [pltpu.VMEM((K, C), jnp.int32)] * 2
            + [pltpu.VMEM((C, F), jnp.float32)] * (2 * NB)
            + [pltpu.SemaphoreType.DMA] * 4
            + [pltpu.VMEM_SHARED((NP, F), jnp.float32)]
        ),
    )(h, zeros, srcs, dsts)


def _tc_mm(x, W1):
    def body(x_ref, w_ref, h_ref):
        h_ref[...] = jnp.dot(x_ref[...], w_ref[...],
                             preferred_element_type=jnp.float32)

    return pl.pallas_call(
        body,
        grid=(NP // BR,),
        in_specs=[
            pl.BlockSpec((BR, 128), lambda r: (r, 0)),
            pl.BlockSpec((128, 64), lambda r: (0, 0)),
        ],
        out_specs=pl.BlockSpec((BR, 64), lambda r: (r, 0)),
        out_shape=jax.ShapeDtypeStruct((NP, 64), jnp.float32),
    )(x, W1)


def _tc_scale(hr, dp):
    def body(hr_ref, dp_ref, h_ref, dis_ref):
        deg = dp_ref[0] + dp_ref[1]
        rid = pl.program_id(0) * BR + lax.broadcasted_iota(jnp.int32, (BR, 1), 0)
        dis = jnp.where(rid < N, lax.rsqrt(deg + 1.0), 0.0)
        dis_ref[...] = dis
        h_ref[...] = hr_ref[...] * dis

    return pl.pallas_call(
        body,
        grid=(NP // BR,),
        in_specs=[
            pl.BlockSpec((BR, 64), lambda r: (r, 0)),
            pl.BlockSpec((2, BR, 1), lambda r: (0, r, 0)),
        ],
        out_specs=[
            pl.BlockSpec((BR, 64), lambda r: (r, 0)),
            pl.BlockSpec((BR, 1), lambda r: (r, 0)),
        ],
        out_shape=[
            jax.ShapeDtypeStruct((NP, 64), jnp.float32),
            jax.ShapeDtypeStruct((NP, 1), jnp.float32),
        ],
    )(hr, dp)


def _tc_mid(parts, dis, b, W, F, Fn):
    def body(p_ref, dis_ref, b_ref, w_ref, h_ref):
        dis = dis_ref[...]
        a = jax.nn.relu((p_ref[0] + p_ref[1]) * dis + b_ref[...])
        h = jnp.dot(a, w_ref[...], preferred_element_type=jnp.float32)
        h_ref[...] = h * dis

    return pl.pallas_call(
        body,
        grid=(NP // BR,),
        in_specs=[
            pl.BlockSpec((2, BR, F), lambda r: (0, r, 0)),
            pl.BlockSpec((BR, 1), lambda r: (r, 0)),
            pl.BlockSpec((1, F), lambda r: (0, 0)),
            pl.BlockSpec((F, Fn), lambda r: (0, 0)),
        ],
        out_specs=pl.BlockSpec((BR, Fn), lambda r: (r, 0)),
        out_shape=jax.ShapeDtypeStruct((NP, Fn), jnp.float32),
    )(parts, dis, b.reshape(1, F), W)


def _tc_last(parts, dis, b):
    def body(p_ref, dis_ref, b_ref, z_ref):
        z_ref[...] = (p_ref[0] + p_ref[1]) * dis_ref[...] + b_ref[...]

    return pl.pallas_call(
        body,
        grid=(NP // BR,),
        in_specs=[
            pl.BlockSpec((2, BR, 16), lambda r: (0, r, 0)),
            pl.BlockSpec((BR, 1), lambda r: (r, 0)),
            pl.BlockSpec((1, 16), lambda r: (0, 0)),
        ],
        out_specs=pl.BlockSpec((BR, 16), lambda r: (r, 0)),
        out_shape=jax.ShapeDtypeStruct((NP, 16), jnp.float32),
    )(parts, dis, b.reshape(1, 16))


def kernel(x, edge_index, W1, b1, W2, b2, W3, b3):
    src = edge_index[0].astype(jnp.int32)
    dst = edge_index[1].astype(jnp.int32)
    pad = N + jnp.arange(EPAD - E, dtype=jnp.int32) % (NP - N)
    srcs = jnp.concatenate([src, pad]).reshape(NW, K, C)
    dsts = jnp.concatenate([dst, pad]).reshape(NW, K, C)
    iden = jnp.arange(80, dtype=jnp.int32)
    x_pad = jnp.pad(x, ((0, NP - N), (0, 0)))
    z64 = jnp.zeros((NP, 64), jnp.float32)
    z32 = jnp.zeros((NP, 32), jnp.float32)
    z16 = jnp.zeros((NP, 16), jnp.float32)

    h1r = _tc_mm(x_pad, W1)
    deg_parts = _deg_call(dsts, iden)
    dp = deg_parts.reshape(2, NP, 1)
    h1, dis = _tc_scale(h1r, dp)
    p1 = _layer_call(64, 1, h1, z64, srcs, dsts)
    h2 = _tc_mid(p1, dis, b1, W2, 64, 32)
    p2 = _layer_call(32, 2, h2, z32, srcs, dsts)
    h3 = _tc_mid(p2, dis, b2, W3, 32, 16)
    p3 = _layer_call(16, 4, h3, z16, srcs, dsts)
    z = _tc_last(p3, dis, b3)
    return z[:N]


# speedup vs baseline: 1.0226x; 1.0226x over previous
import jax
import jax.numpy as jnp
from jax import lax
from jax.experimental import pallas as pl
from jax.experimental.pallas import tpu as pltpu
from jax.experimental.pallas import tpu_sc as plsc

N = 10000
NP = 10240
E = 640000
NW = 32
C = 256
K = 80
EPW = K * C
EPAD = NW * EPW
RPT = NP // 16
BR = 1280


def _mesh():
    return plsc.VectorSubcoreMesh(core_axis_name="c", subcore_axis_name="s")


def _deg_call(dsts, iden):
    def body(dsts_hbm, iden_hbm, part_hbm, dst_v, deg1_v, deg2_v, iden_v, shared):
        cid = lax.axis_index("c")
        sid = lax.axis_index("s")
        wid = sid * 2 + cid
        pltpu.sync_copy(dsts_hbm.at[wid], dst_v)
        pltpu.sync_copy(iden_hbm, iden_v)

        zero = jnp.zeros((16,), jnp.float32)

        def zbody(i, carry):
            deg1_v[pl.ds(i * 16, 16)] = zero
            return carry

        lax.fori_loop(0, NP // 16, zbody, 0)

        def z2body(t, carry):
            for s in range(8):
                deg2_v[t, pl.ds(s * 16, 16)] = zero
            return carry

        lax.fori_loop(0, 80, z2body, 0)

        @pl.when(sid == 0)
        def _():
            pltpu.sync_copy(deg2_v, shared)

        ones = jnp.ones((16,), jnp.float32)

        def ebody(j, carry):
            for s in range(C // 16):
                idx = dst_v[j, pl.ds(s * 16, 16)]
                plsc.addupdate_scatter(deg1_v, [idx], ones)
            return carry

        lax.fori_loop(0, K, ebody, 0)

        def pbody(t, carry):
            for s in range(8):
                deg2_v[t, pl.ds(s * 16, 16)] = deg1_v[pl.ds(t * 128 + s * 16, 16)]
            return carry

        lax.fori_loop(0, 80, pbody, 0)
        plsc.subcore_barrier()
        pltpu.sync_copy(deg2_v, shared.at[iden_v], add=True)
        plsc.subcore_barrier()

        @pl.when(sid < 10)
        def _():
            sl = pl.ds(sid * 8, 8)
            pltpu.sync_copy(shared.at[sl], part_hbm.at[cid, sl])

    return pl.kernel(
        body,
        out_type=jax.ShapeDtypeStruct((2, 80, 128), jnp.float32),
        mesh=_mesh(),
        compiler_params=pltpu.CompilerParams(
            needs_layout_passes=False, use_tc_tiling_on_sc=False),
        scratch_types=[
            pltpu.VMEM((K, C), jnp.int32),
            pltpu.VMEM((NP,), jnp.float32),
            pltpu.VMEM((80, 128), jnp.float32),
            pltpu.VMEM((80,), jnp.int32),
            pltpu.VMEM_SHARED((80, 128), jnp.float32),
        ],
    )(dsts, iden)


def _layer_call(F, NB, h, zeros, srcs, dsts):
    KB = K // NB

    def body(*refs):
        h_hbm, z_hbm, srcs_hbm, dsts_hbm, part_hbm, src_v, dst_v = refs[:7]
        bufs = refs[7:7 + 2 * NB]
        gsem = refs[7 + 2 * NB:9 + 2 * NB]
        ssem = refs[9 + 2 * NB:11 + 2 * NB]
        xsem, isem = refs[11 + 2 * NB:13 + 2 * NB]
        shared = refs[13 + 2 * NB]
        rows = (bufs[:NB], bufs[NB:])
        cid = lax.axis_index("c")
        sid = lax.axis_index("s")
        wid = sid * 2 + cid
        sl = pl.ds(sid * RPT, RPT)

        pltpu.async_copy(srcs_hbm.at[wid], src_v, xsem)
        pltpu.async_copy(dsts_hbm.at[wid], dst_v, xsem)

        @pl.when(cid == 0)
        def _():
            pltpu.async_copy(h_hbm.at[sl], shared.at[sl], isem)

        @pl.when(cid != 0)
        def _():
            pltpu.async_copy(z_hbm.at[sl], shared.at[sl], isem)

        pltpu.make_async_copy(srcs_hbm.at[wid], src_v, xsem).wait()
        pltpu.make_async_copy(dsts_hbm.at[wid], dst_v, xsem).wait()

        dummy = h_hbm.at[pl.ds(0, C)]

        def start_gathers(t, p):
            for b in range(NB):
                pltpu.async_copy(h_hbm.at[src_v.at[t * NB + b]], rows[p][b], gsem[p])

        def fire_scatters(t, p):
            for b in range(NB):
                pltpu.async_copy(rows[p][b], shared.at[dst_v.at[t * NB + b]],
                                 ssem[p], add=True)

        def drain(sem, p):
            for b in range(NB):
                pltpu.make_async_copy(dummy, rows[p][b], sem).wait()

        start_gathers(0, 0)
        pltpu.make_async_copy(z_hbm.at[sl], shared.at[sl], isem).wait()
        plsc.subcore_barrier()

        def phase(t, p):
            @pl.when(t > 0)
            def _():
                drain(ssem[1 - p], 1 - p)

            @pl.when(t + 1 < KB)
            def _():
                start_gathers(t + 1, 1 - p)

            drain(gsem[p], p)
            fire_scatters(t, p)

        def outer(i, carry):
            phase(2 * i, 0)
            phase(2 * i + 1, 1)
            return carry

        lax.fori_loop(0, KB // 2, outer, 0)
        drain(ssem[(KB - 1) % 2], (KB - 1) % 2)
        plsc.subcore_barrier()
        wrpt = RPT // 4
        for w in range(4):
            wsl = pl.ds(sid * RPT + w * wrpt, wrpt)
            pltpu.async_copy(shared.at[wsl], part_hbm.at[cid, wsl], isem)
        for w in range(4):
            wsl = pl.ds(sid * RPT + w * wrpt, wrpt)
            pltpu.make_async_copy(shared.at[wsl], part_hbm.at[cid, wsl], isem).wait()

    return pl.kernel(
        body,
        out_type=jax.ShapeDtypeStruct((2, NP, F), jnp.float32),
        mesh=_mesh(),
        compiler_params=pltpu.CompilerParams(use_tc_tiling_on_sc=False),
        scratch_types=(
            [pltpu.VMEM((K, C), jnp.int32)] * 2
            + [pltpu.VMEM((C, F), jnp.float32)] * (2 * NB)
            + [pltpu.SemaphoreType.DMA] * 6
            + [pltpu.VMEM_SHARED((NP, F), jnp.float32)]
        ),
    )(h, zeros, srcs, dsts)


def _tc_mm(x, W1):
    def body(x_ref, w_ref, h_ref):
        h_ref[...] = jnp.dot(x_ref[...], w_ref[...],
                             preferred_element_type=jnp.float32)

    return pl.pallas_call(
        body,
        grid=(NP // BR,),
        in_specs=[
            pl.BlockSpec((BR, 128), lambda r: (r, 0)),
            pl.BlockSpec((128, 64), lambda r: (0, 0)),
        ],
        out_specs=pl.BlockSpec((BR, 64), lambda r: (r, 0)),
        out_shape=jax.ShapeDtypeStruct((NP, 64), jnp.float32),
    )(x, W1)


def _tc_scale(hr, dp):
    def body(hr_ref, dp_ref, h_ref, dis_ref):
        deg = dp_ref[0] + dp_ref[1]
        rid = pl.program_id(0) * BR + lax.broadcasted_iota(jnp.int32, (BR, 1), 0)
        dis = jnp.where(rid < N, lax.rsqrt(deg + 1.0), 0.0)
        dis_ref[...] = dis
        h_ref[...] = hr_ref[...] * dis

    return pl.pallas_call(
        body,
        grid=(NP // BR,),
        in_specs=[
            pl.BlockSpec((BR, 64), lambda r: (r, 0)),
            pl.BlockSpec((2, BR, 1), lambda r: (0, r, 0)),
        ],
        out_specs=[
            pl.BlockSpec((BR, 64), lambda r: (r, 0)),
            pl.BlockSpec((BR, 1), lambda r: (r, 0)),
        ],
        out_shape=[
            jax.ShapeDtypeStruct((NP, 64), jnp.float32),
            jax.ShapeDtypeStruct((NP, 1), jnp.float32),
        ],
    )(hr, dp)


def _tc_mid(parts, dis, b, W, F, Fn):
    def body(p_ref, dis_ref, b_ref, w_ref, h_ref):
        dis = dis_ref[...]
        a = jax.nn.relu((p_ref[0] + p_ref[1]) * dis + b_ref[...])
        h = jnp.dot(a, w_ref[...], preferred_element_type=jnp.float32)
        h_ref[...] = h * dis

    return pl.pallas_call(
        body,
        grid=(NP // BR,),
        in_specs=[
            pl.BlockSpec((2, BR, F), lambda r: (0, r, 0)),
            pl.BlockSpec((BR, 1), lambda r: (r, 0)),
            pl.BlockSpec((1, F), lambda r: (0, 0)),
            pl.BlockSpec((F, Fn), lambda r: (0, 0)),
        ],
        out_specs=pl.BlockSpec((BR, Fn), lambda r: (r, 0)),
        out_shape=jax.ShapeDtypeStruct((NP, Fn), jnp.float32),
    )(parts, dis, b.reshape(1, F), W)


def _tc_last(parts, dis, b):
    def body(p_ref, dis_ref, b_ref, z_ref):
        z_ref[...] = (p_ref[0] + p_ref[1]) * dis_ref[...] + b_ref[...]

    return pl.pallas_call(
        body,
        grid=(NP // BR,),
        in_specs=[
            pl.BlockSpec((2, BR, 16), lambda r: (0, r, 0)),
            pl.BlockSpec((BR, 1), lambda r: (r, 0)),
            pl.BlockSpec((1, 16), lambda r: (0, 0)),
        ],
        out_specs=pl.BlockSpec((BR, 16), lambda r: (r, 0)),
        out_shape=jax.ShapeDtypeStruct((NP, 16), jnp.float32),
    )(parts, dis, b.reshape(1, 16))


def kernel(x, edge_index, W1, b1, W2, b2, W3, b3):
    src = edge_index[0].astype(jnp.int32)
    dst = edge_index[1].astype(jnp.int32)
    pad = N + jnp.arange(EPAD - E, dtype=jnp.int32) % (NP - N)
    srcs = jnp.concatenate([src, pad]).reshape(NW, K, C)
    dsts = jnp.concatenate([dst, pad]).reshape(NW, K, C)
    iden = jnp.arange(80, dtype=jnp.int32)
    x_pad = jnp.pad(x, ((0, NP - N), (0, 0)))
    z64 = jnp.zeros((NP, 64), jnp.float32)
    z32 = jnp.zeros((NP, 32), jnp.float32)
    z16 = jnp.zeros((NP, 16), jnp.float32)

    h1r = _tc_mm(x_pad, W1)
    deg_parts = _deg_call(dsts, iden)
    dp = deg_parts.reshape(2, NP, 1)
    h1, dis = _tc_scale(h1r, dp)
    p1 = _layer_call(64, 1, h1, z64, srcs, dsts)
    h2 = _tc_mid(p1, dis, b1, W2, 64, 32)
    p2 = _layer_call(32, 2, h2, z32, srcs, dsts)
    h3 = _tc_mid(p2, dis, b2, W3, 32, 16)
    p3 = _layer_call(16, 4, h3, z16, srcs, dsts)
    z = _tc_last(p3, dis, b3)
    return z[:N]
